# baseline (speedup 1.0000x reference)
import functools

import jax
import jax.numpy as jnp
from jax import lax
from jax.experimental import pallas as pl
from jax.experimental.pallas import tpu as pltpu
from jax.experimental.pallas import tpu_sc as plsc

N_NODES = 10000
N_EDGES = 320000
D_FEAT = 128
NHID = 128
LATENT = 64

NC = 2
NS = 16
NW = NC * NS
CH = 128

NPAD = 10240
STRIPE = NPAD // NS
DUMMY = NPAD - 8

NCH = 80
NBUF = 4
EW = NCH * CH
EPAD = EW * NW


@functools.lru_cache(maxsize=1)
def _sc_kernels():
    mesh = plsc.VectorSubcoreMesh(core_axis_name="c", subcore_axis_name="s",
                                  num_cores=NC, num_subcores=NS)

    @functools.partial(
        pl.kernel,
        out_type=jax.ShapeDtypeStruct((NC, NPAD, D_FEAT), jnp.float32),
        mesh=mesh,
        scratch_types=[
            pltpu.VMEM((NCH, CH), jnp.int32),
            pltpu.VMEM((CH, D_FEAT), jnp.float32),
            pltpu.VMEM_SHARED((NPAD, D_FEAT), jnp.float32),
        ],
    )
    def hist(dst_hbm, ones_hbm, zeros_hbm, out_hbm, idxd_v, ones_v, acc_sh):
        c = lax.axis_index("c")
        s = lax.axis_index("s")
        pltpu.sync_copy(zeros_hbm, acc_sh.at[pl.ds(s * STRIPE, STRIPE)])
        pltpu.sync_copy(ones_hbm, ones_v)
        w = c * NS + s
        pltpu.sync_copy(dst_hbm.at[pl.ds(w * NCH, NCH)], idxd_v)
        plsc.subcore_barrier()

        def outer(k, carry):
            pltpu.sync_copy(ones_v, acc_sh.at[idxd_v.at[k]], add=True)
            return carry

        lax.fori_loop(0, NCH, outer, 0)
        plsc.subcore_barrier()
        pltpu.sync_copy(acc_sh.at[pl.ds(s * STRIPE, STRIPE)],
                        out_hbm.at[c, pl.ds(s * STRIPE, STRIPE)])

    @functools.partial(
        pl.kernel,
        out_type=jax.ShapeDtypeStruct((NC, NPAD, D_FEAT), jnp.float32),
        mesh=mesh,
        scratch_types=[
            pltpu.VMEM((NCH // 2, CH), jnp.int32),
            pltpu.VMEM((NCH // 2, CH), jnp.int32),
            pltpu.VMEM((2, CH, D_FEAT), jnp.float32),
            pltpu.VMEM_SHARED((NPAD, D_FEAT), jnp.float32),
            pltpu.SemaphoreType.DMA,
            pltpu.SemaphoreType.DMA,
        ],
    )
    def prop(src_hbm, dst_hbm, table_hbm, zeros_hbm, out_hbm,
             idxs_v, idxd_v, rows3, acc_sh, sem0, sem1):
        rows = [rows3.at[0], rows3.at[1]]
        sems = [sem0, sem1]
        c = lax.axis_index("c")
        s = lax.axis_index("s")
        pltpu.sync_copy(zeros_hbm, acc_sh.at[pl.ds(s * STRIPE, STRIPE)])
        w = c * NS + s
        plsc.subcore_barrier()
        HCH = NCH // 2

        def half(hh, carry):
            hbase = w * NCH + hh * HCH
            pltpu.sync_copy(src_hbm.at[pl.ds(hbase, HCH)], idxs_v)
            pltpu.sync_copy(dst_hbm.at[pl.ds(hbase, HCH)], idxd_v)
            for b in range(2):
                pltpu.async_copy(table_hbm.at[idxs_v.at[b]], rows[b], sems[b])

            def outer(g, carry2):
                for b in range(2):
                    k = g * 2 + b
                    pltpu.make_async_copy(
                        table_hbm.at[idxs_v.at[k]], rows[b], sems[b]).wait()
                    pltpu.sync_copy(rows[b], acc_sh.at[idxd_v.at[k]], add=True)

                    @pl.when(k + 2 < HCH)
                    def _next():
                        pltpu.async_copy(
                            table_hbm.at[idxs_v.at[k + 2]], rows[b], sems[b])
                return carry2

            lax.fori_loop(0, HCH // 2, outer, 0)
            return carry

        lax.fori_loop(0, 2, half, 0)
        plsc.subcore_barrier()
        pltpu.sync_copy(acc_sh.at[pl.ds(s * STRIPE, STRIPE)],
                        out_hbm.at[c, pl.ds(s * STRIPE, STRIPE)])

    return hist, prop


BR = 1024


def _mm1_body(x_ref, w_ref, deg_ref, h1_ref, u1_ref, dinv_ref):
    h1 = jnp.dot(x_ref[...], w_ref[...], preferred_element_type=jnp.float32)
    deg = deg_ref[0, :, 0] + deg_ref[1, :, 0] + 1.0
    dinv = lax.rsqrt(deg)
    h1_ref[...] = h1
    u1_ref[...] = h1 * dinv[:, None]
    dinv_ref[...] = dinv


def _tc_mm1(x_p, W1, deg_parts):
    grid = (NPAD // BR,)
    return pl.pallas_call(
        _mm1_body,
        grid=grid,
        in_specs=[
            pl.BlockSpec((BR, D_FEAT), lambda i: (i, 0)),
            pl.BlockSpec((D_FEAT, NHID), lambda i: (0, 0)),
            pl.BlockSpec((NC, BR, D_FEAT), lambda i: (0, i, 0)),
        ],
        out_specs=[
            pl.BlockSpec((BR, NHID), lambda i: (i, 0)),
            pl.BlockSpec((BR, NHID), lambda i: (i, 0)),
            pl.BlockSpec((BR,), lambda i: (i,)),
        ],
        out_shape=[
            jax.ShapeDtypeStruct((NPAD, NHID), jnp.float32),
            jax.ShapeDtypeStruct((NPAD, NHID), jnp.float32),
            jax.ShapeDtypeStruct((NPAD,), jnp.float32),
        ],
    )(x_p, W1, deg_parts)


def _mid_body(p_ref, dinv_ref, h1_ref, b1_ref, h_ref, u2_ref):
    dv = dinv_ref[...][:, None]
    t = (p_ref[0] + p_ref[1]) * dv + dv * dv * h1_ref[...] + b1_ref[...]
    h = jnp.maximum(t, 0.0)
    h_ref[...] = h
    u2_ref[...] = h * dv


def _tc_mid(p1, dinv, h1, b1):
    grid = (NPAD // BR,)
    return pl.pallas_call(
        _mid_body,
        grid=grid,
        in_specs=[
            pl.BlockSpec((NC, BR, NHID), lambda i: (0, i, 0)),
            pl.BlockSpec((BR,), lambda i: (i,)),
            pl.BlockSpec((BR, NHID), lambda i: (i, 0)),
            pl.BlockSpec((NHID,), lambda i: (0,)),
        ],
        out_specs=[
            pl.BlockSpec((BR, NHID), lambda i: (i, 0)),
            pl.BlockSpec((BR, NHID), lambda i: (i, 0)),
        ],
        out_shape=[
            jax.ShapeDtypeStruct((NPAD, NHID), jnp.float32),
            jax.ShapeDtypeStruct((NPAD, NHID), jnp.float32),
        ],
    )(p1, dinv, h1, b1)


def _fin_body(p_ref, dinv_ref, h_ref, wm_ref, bm_ref, wl_ref, bl_ref,
              mu_ref, lv_ref):
    dv = dinv_ref[...][:, None]
    g = (p_ref[0] + p_ref[1]) * dv + dv * dv * h_ref[...]
    mu_ref[...] = jnp.dot(g, wm_ref[...], preferred_element_type=jnp.float32) + bm_ref[...]
    lv_ref[...] = jnp.dot(g, wl_ref[...], preferred_element_type=jnp.float32) + bl_ref[...]


def _tc_fin(p2, dinv, h, W_mu, b_mu, W_logvar, b_logvar):
    grid = (NPAD // BR,)
    return pl.pallas_call(
        _fin_body,
        grid=grid,
        in_specs=[
            pl.BlockSpec((NC, BR, NHID), lambda i: (0, i, 0)),
            pl.BlockSpec((BR,), lambda i: (i,)),
            pl.BlockSpec((BR, NHID), lambda i: (i, 0)),
            pl.BlockSpec((NHID, LATENT), lambda i: (0, 0)),
            pl.BlockSpec((LATENT,), lambda i: (0,)),
            pl.BlockSpec((NHID, LATENT), lambda i: (0, 0)),
            pl.BlockSpec((LATENT,), lambda i: (0,)),
        ],
        out_specs=[
            pl.BlockSpec((BR, LATENT), lambda i: (i, 0)),
            pl.BlockSpec((BR, LATENT), lambda i: (i, 0)),
        ],
        out_shape=[
            jax.ShapeDtypeStruct((NPAD, LATENT), jnp.float32),
            jax.ShapeDtypeStruct((NPAD, LATENT), jnp.float32),
        ],
    )(p2, dinv, h, W_mu, b_mu, W_logvar, b_logvar)


@jax.jit
def kernel(x, edge_index, W1, b1, W_mu, b_mu, W_logvar, b_logvar):
    src = edge_index[0]
    dst = edge_index[1]
    pad = EPAD - N_EDGES
    src_p = jnp.concatenate([src, jnp.zeros((pad,), jnp.int32)]).reshape(
        NW * NCH, CH)
    dst_p = jnp.concatenate([dst, jnp.full((pad,), DUMMY, jnp.int32)]).reshape(
        NW * NCH, CH)
    x_p = jnp.pad(x, ((0, NPAD - N_NODES), (0, 0)))

    zerosD = jnp.zeros((STRIPE, D_FEAT), jnp.float32)

    hist, prop = _sc_kernels()
    onesD = jnp.ones((CH, D_FEAT), jnp.float32)
    deg_parts = hist(dst_p, onesD, zerosD)
    h1, u1, dinv = _tc_mm1(x_p, W1, deg_parts)
    p1 = prop(src_p, dst_p, u1, zerosD)
    h, u2 = _tc_mid(p1, dinv, h1, b1)
    p2 = prop(src_p, dst_p, u2, zerosD)
    mu, logvar = _tc_fin(p2, dinv, h, W_mu, b_mu, W_logvar, b_logvar)
    return mu[:N_NODES], logvar[:N_NODES]


# speedup vs baseline: 1.0724x; 1.0724x over previous
import functools

import jax
import jax.numpy as jnp
from jax import lax
from jax.experimental import pallas as pl
from jax.experimental.pallas import tpu as pltpu
from jax.experimental.pallas import tpu_sc as plsc

N_NODES = 10000
N_EDGES = 320000
D_FEAT = 128
NHID = 128
LATENT = 64

NC = 2
NS = 16
NW = NC * NS
CH = 128

NPAD = 10240
STRIPE = NPAD // NS
DUMMY = NPAD - 8

NCH = 80
TCH = NW * NCH
BS = 32
NB0 = 4
NB1 = 1
EPAD = TCH * CH


@functools.lru_cache(maxsize=1)
def _sc_kernels():
    mesh = plsc.VectorSubcoreMesh(core_axis_name="c", subcore_axis_name="s",
                                  num_cores=NC, num_subcores=NS)

    @functools.partial(
        pl.kernel,
        out_type=jax.ShapeDtypeStruct((NC, NPAD, D_FEAT), jnp.float32),
        mesh=mesh,
        scratch_types=[
            pltpu.VMEM((NCH, CH), jnp.int32),
            pltpu.VMEM((CH, D_FEAT), jnp.float32),
            pltpu.VMEM_SHARED((NPAD, D_FEAT), jnp.float32),
        ],
    )
    def hist(dst_hbm, ones_hbm, zeros_hbm, out_hbm, idxd_v, ones_v, acc_sh):
        c = lax.axis_index("c")
        s = lax.axis_index("s")
        pltpu.sync_copy(zeros_hbm, acc_sh.at[pl.ds(s * STRIPE, STRIPE)])
        pltpu.sync_copy(ones_hbm, ones_v)
        w = c * NS + s
        pltpu.sync_copy(dst_hbm.at[pl.ds(w * NCH, NCH)], idxd_v)
        plsc.subcore_barrier()

        def outer(k, carry):
            pltpu.sync_copy(ones_v, acc_sh.at[idxd_v.at[k]], add=True)
            return carry

        lax.fori_loop(0, NCH, outer, 0)
        plsc.subcore_barrier()
        pltpu.sync_copy(acc_sh.at[pl.ds(s * STRIPE, STRIPE)],
                        out_hbm.at[c, pl.ds(s * STRIPE, STRIPE)])

    @functools.partial(
        pl.kernel,
        out_type=jax.ShapeDtypeStruct((NC, NPAD, D_FEAT), jnp.float32),
        mesh=mesh,
        scratch_types=[
            pltpu.VMEM((BS, CH), jnp.int32),
            pltpu.VMEM((BS, CH), jnp.int32),
            pltpu.VMEM((2, CH, D_FEAT), jnp.float32),
            pltpu.VMEM_SHARED((NPAD, D_FEAT), jnp.float32),
            pltpu.SemaphoreType.DMA,
            pltpu.SemaphoreType.DMA,
        ],
    )
    def prop(src_hbm, dst_hbm, table_hbm, zeros_hbm, out_hbm,
             idxs_v, idxd_v, rows3, acc_sh, sem0, sem1):
        rows = [rows3.at[0], rows3.at[1]]
        sems = [sem0, sem1]
        c = lax.axis_index("c")
        s = lax.axis_index("s")
        pltpu.sync_copy(zeros_hbm, acc_sh.at[pl.ds(s * STRIPE, STRIPE)])
        nb = jnp.where(c == 0, NB0, NB1)
        chunk0 = jnp.where(c == 0, s * (NB0 * BS), NS * NB0 * BS + s * (NB1 * BS))
        plsc.subcore_barrier()

        def block(bb, carry):
            bbase = chunk0 + bb * BS
            pltpu.sync_copy(src_hbm.at[pl.ds(bbase, BS)], idxs_v)
            pltpu.sync_copy(dst_hbm.at[pl.ds(bbase, BS)], idxd_v)
            for b in range(2):
                pltpu.async_copy(table_hbm.at[idxs_v.at[b]], rows[b], sems[b])

            def outer(g, carry2):
                for b in range(2):
                    k = g * 2 + b
                    pltpu.make_async_copy(
                        table_hbm.at[idxs_v.at[k]], rows[b], sems[b]).wait()
                    pltpu.sync_copy(rows[b], acc_sh.at[idxd_v.at[k]], add=True)

                    @pl.when(k + 2 < BS)
                    def _next():
                        pltpu.async_copy(
                            table_hbm.at[idxs_v.at[k + 2]], rows[b], sems[b])
                return carry2

            lax.fori_loop(0, BS // 2, outer, 0)
            return carry

        lax.fori_loop(0, nb, block, 0)
        plsc.subcore_barrier()
        pltpu.sync_copy(acc_sh.at[pl.ds(s * STRIPE, STRIPE)],
                        out_hbm.at[c, pl.ds(s * STRIPE, STRIPE)])

    return hist, prop


BR = 1024


def _mm1_body(x_ref, w_ref, deg_ref, h1_ref, u1_ref, dinv_ref):
    h1 = jnp.dot(x_ref[...], w_ref[...], preferred_element_type=jnp.float32)
    deg = deg_ref[0, :, 0] + deg_ref[1, :, 0] + 1.0
    dinv = lax.rsqrt(deg)
    h1_ref[...] = h1
    u1_ref[...] = h1 * dinv[:, None]
    dinv_ref[...] = dinv


def _tc_mm1(x_p, W1, deg_parts):
    grid = (NPAD // BR,)
    return pl.pallas_call(
        _mm1_body,
        grid=grid,
        in_specs=[
            pl.BlockSpec((BR, D_FEAT), lambda i: (i, 0)),
            pl.BlockSpec((D_FEAT, NHID), lambda i: (0, 0)),
            pl.BlockSpec((NC, BR, D_FEAT), lambda i: (0, i, 0)),
        ],
        out_specs=[
            pl.BlockSpec((BR, NHID), lambda i: (i, 0)),
            pl.BlockSpec((BR, NHID), lambda i: (i, 0)),
            pl.BlockSpec((BR,), lambda i: (i,)),
        ],
        out_shape=[
            jax.ShapeDtypeStruct((NPAD, NHID), jnp.float32),
            jax.ShapeDtypeStruct((NPAD, NHID), jnp.float32),
            jax.ShapeDtypeStruct((NPAD,), jnp.float32),
        ],
    )(x_p, W1, deg_parts)


def _mid_body(p_ref, dinv_ref, h1_ref, b1_ref, h_ref, u2_ref):
    dv = dinv_ref[...][:, None]
    t = (p_ref[0] + p_ref[1]) * dv + dv * dv * h1_ref[...] + b1_ref[...]
    h = jnp.maximum(t, 0.0)
    h_ref[...] = h
    u2_ref[...] = h * dv


def _tc_mid(p1, dinv, h1, b1):
    grid = (NPAD // BR,)
    return pl.pallas_call(
        _mid_body,
        grid=grid,
        in_specs=[
            pl.BlockSpec((NC, BR, NHID), lambda i: (0, i, 0)),
            pl.BlockSpec((BR,), lambda i: (i,)),
            pl.BlockSpec((BR, NHID), lambda i: (i, 0)),
            pl.BlockSpec((NHID,), lambda i: (0,)),
        ],
        out_specs=[
            pl.BlockSpec((BR, NHID), lambda i: (i, 0)),
            pl.BlockSpec((BR, NHID), lambda i: (i, 0)),
        ],
        out_shape=[
            jax.ShapeDtypeStruct((NPAD, NHID), jnp.float32),
            jax.ShapeDtypeStruct((NPAD, NHID), jnp.float32),
        ],
    )(p1, dinv, h1, b1)


def _fin_body(p_ref, dinv_ref, h_ref, wm_ref, bm_ref, wl_ref, bl_ref,
              mu_ref, lv_ref):
    dv = dinv_ref[...][:, None]
    g = (p_ref[0] + p_ref[1]) * dv + dv * dv * h_ref[...]
    mu_ref[...] = jnp.dot(g, wm_ref[...], preferred_element_type=jnp.float32) + bm_ref[...]
    lv_ref[...] = jnp.dot(g, wl_ref[...], preferred_element_type=jnp.float32) + bl_ref[...]


def _tc_fin(p2, dinv, h, W_mu, b_mu, W_logvar, b_logvar):
    grid = (NPAD // BR,)
    return pl.pallas_call(
        _fin_body,
        grid=grid,
        in_specs=[
            pl.BlockSpec((NC, BR, NHID), lambda i: (0, i, 0)),
            pl.BlockSpec((BR,), lambda i: (i,)),
            pl.BlockSpec((BR, NHID), lambda i: (i, 0)),
            pl.BlockSpec((NHID, LATENT), lambda i: (0, 0)),
            pl.BlockSpec((LATENT,), lambda i: (0,)),
            pl.BlockSpec((NHID, LATENT), lambda i: (0, 0)),
            pl.BlockSpec((LATENT,), lambda i: (0,)),
        ],
        out_specs=[
            pl.BlockSpec((BR, LATENT), lambda i: (i, 0)),
            pl.BlockSpec((BR, LATENT), lambda i: (i, 0)),
        ],
        out_shape=[
            jax.ShapeDtypeStruct((NPAD, LATENT), jnp.float32),
            jax.ShapeDtypeStruct((NPAD, LATENT), jnp.float32),
        ],
    )(p2, dinv, h, W_mu, b_mu, W_logvar, b_logvar)


@jax.jit
def kernel(x, edge_index, W1, b1, W_mu, b_mu, W_logvar, b_logvar):
    src = edge_index[0]
    dst = edge_index[1]
    pad = EPAD - N_EDGES
    src_p = jnp.concatenate([src, jnp.zeros((pad,), jnp.int32)]).reshape(
        NW * NCH, CH)
    dst_p = jnp.concatenate([dst, jnp.full((pad,), DUMMY, jnp.int32)]).reshape(
        NW * NCH, CH)
    x_p = jnp.pad(x, ((0, NPAD - N_NODES), (0, 0)))

    zerosD = jnp.zeros((STRIPE, D_FEAT), jnp.float32)

    hist, prop = _sc_kernels()
    onesD = jnp.ones((CH, D_FEAT), jnp.float32)
    deg_parts = hist(dst_p, onesD, zerosD)
    h1, u1, dinv = _tc_mm1(x_p, W1, deg_parts)
    p1 = prop(src_p, dst_p, u1, zerosD)
    h, u2 = _tc_mid(p1, dinv, h1, b1)
    p2 = prop(src_p, dst_p, u2, zerosD)
    mu, logvar = _tc_fin(p2, dinv, h, W_mu, b_mu, W_logvar, b_logvar)
    return mu[:N_NODES], logvar[:N_NODES]


# speedup vs baseline: 2.9571x; 2.7575x over previous
import functools

import jax
import jax.numpy as jnp
from jax import lax
from jax.experimental import pallas as pl
from jax.experimental.pallas import tpu as pltpu
from jax.experimental.pallas import tpu_sc as plsc

N_NODES = 10000
N_EDGES = 320000
D_FEAT = 128
NHID = 128
LATENT = 64

NC = 2
NS = 16
NW = NC * NS
CH = 128

NPAD = 10240
STRIPE = NPAD // NS
DUMMY = NPAD - 8

NCH = 80
TCH = NW * NCH
BS = 40
NB = 2
EPAD = TCH * CH


@functools.lru_cache(maxsize=1)
def _sc_kernels():
    mesh = plsc.VectorSubcoreMesh(core_axis_name="c", subcore_axis_name="s",
                                  num_cores=NC, num_subcores=NS)

    @functools.partial(
        pl.kernel,
        out_type=jax.ShapeDtypeStruct((NC, NPAD, D_FEAT), jnp.float32),
        mesh=mesh,
        scratch_types=[
            pltpu.VMEM((NCH, CH), jnp.int32),
            pltpu.VMEM((CH, D_FEAT), jnp.float32),
            pltpu.VMEM_SHARED((NPAD, D_FEAT), jnp.float32),
        ],
    )
    def hist(dst_hbm, ones_hbm, zeros_hbm, out_hbm, idxd_v, ones_v, acc_sh):
        c = lax.axis_index("c")
        s = lax.axis_index("s")
        pltpu.sync_copy(zeros_hbm, acc_sh.at[pl.ds(s * STRIPE, STRIPE)])
        pltpu.sync_copy(ones_hbm, ones_v)
        w = c * NS + s
        pltpu.sync_copy(dst_hbm.at[pl.ds(w * NCH, NCH)], idxd_v)
        plsc.subcore_barrier()

        def outer(k, carry):
            pltpu.sync_copy(ones_v, acc_sh.at[idxd_v.at[k]], add=True)
            return carry

        lax.fori_loop(0, NCH, outer, 0)
        plsc.subcore_barrier()
        pltpu.sync_copy(acc_sh.at[pl.ds(s * STRIPE, STRIPE)],
                        out_hbm.at[c, pl.ds(s * STRIPE, STRIPE)])

    @functools.partial(
        pl.kernel,
        out_type=jax.ShapeDtypeStruct((NC, NPAD, D_FEAT), jnp.float32),
        mesh=mesh,
        scratch_types=[
            pltpu.VMEM((BS, CH), jnp.int32),
            pltpu.VMEM((BS, CH), jnp.int32) ,
            pltpu.VMEM((2, CH, D_FEAT), jnp.float32),
            pltpu.VMEM_SHARED((NPAD, D_FEAT), jnp.float32),
            pltpu.SemaphoreType.DMA,
            pltpu.SemaphoreType.DMA,
        ],
    )
    def prop(src_hbm, dst_hbm, table_hbm, zeros_hbm, out_hbm,
             idxs_v, idxd_v, rows3, acc_sh, sem0, sem1):
        rows = [rows3.at[0], rows3.at[1]]
        sems = [sem0, sem1]
        c = lax.axis_index("c")
        s = lax.axis_index("s")
        pltpu.sync_copy(zeros_hbm, acc_sh.at[pl.ds(s * STRIPE, STRIPE)])
        chunk0 = (c * NS + s) * (NB * BS)
        plsc.subcore_barrier()

        def block(bb, carry):
            bbase = chunk0 + bb * BS
            pltpu.sync_copy(src_hbm.at[pl.ds(bbase, BS)], idxs_v)
            pltpu.sync_copy(dst_hbm.at[pl.ds(bbase, BS)], idxd_v)
            for b in range(2):
                pltpu.async_copy(table_hbm.at[idxs_v.at[b]], rows[b], sems[b])

            def outer(g, carry2):
                for b in range(2):
                    k = g * 2 + b
                    pltpu.make_async_copy(
                        table_hbm.at[idxs_v.at[k]], rows[b], sems[b]).wait()
                    pltpu.sync_copy(rows[b], acc_sh.at[idxd_v.at[k]], add=True)

                    @pl.when(k + 2 < BS)
                    def _next():
                        pltpu.async_copy(
                            table_hbm.at[idxs_v.at[k + 2]], rows[b], sems[b])
                return carry2

            lax.fori_loop(0, BS // 2, outer, 0)
            return carry

        lax.fori_loop(0, NB, block, 0)
        plsc.subcore_barrier()
        pltpu.sync_copy(acc_sh.at[pl.ds(s * STRIPE, STRIPE)],
                        out_hbm.at[c, pl.ds(s * STRIPE, STRIPE)])

    return hist, prop


BR = 1024


def _mm1_body(x_ref, w_ref, deg_ref, h1_ref, u1_ref, dinv_ref):
    h1 = jnp.dot(x_ref[...], w_ref[...], preferred_element_type=jnp.float32)
    deg = deg_ref[0, :, 0] + deg_ref[1, :, 0] + 1.0
    dinv = lax.rsqrt(deg)
    h1_ref[...] = h1
    u1_ref[...] = h1 * dinv[:, None]
    dinv_ref[...] = dinv


def _tc_mm1(x_p, W1, deg_parts):
    grid = (NPAD // BR,)
    return pl.pallas_call(
        _mm1_body,
        grid=grid,
        in_specs=[
            pl.BlockSpec((BR, D_FEAT), lambda i: (i, 0)),
            pl.BlockSpec((D_FEAT, NHID), lambda i: (0, 0)),
            pl.BlockSpec((NC, BR, D_FEAT), lambda i: (0, i, 0)),
        ],
        out_specs=[
            pl.BlockSpec((BR, NHID), lambda i: (i, 0)),
            pl.BlockSpec((BR, NHID), lambda i: (i, 0)),
            pl.BlockSpec((BR,), lambda i: (i,)),
        ],
        out_shape=[
            jax.ShapeDtypeStruct((NPAD, NHID), jnp.float32),
            jax.ShapeDtypeStruct((NPAD, NHID), jnp.float32),
            jax.ShapeDtypeStruct((NPAD,), jnp.float32),
        ],
    )(x_p, W1, deg_parts)


def _mid_body(p_ref, dinv_ref, h1_ref, b1_ref, h_ref, u2_ref):
    dv = dinv_ref[...][:, None]
    t = (p_ref[0] + p_ref[1]) * dv + dv * dv * h1_ref[...] + b1_ref[...]
    h = jnp.maximum(t, 0.0)
    h_ref[...] = h
    u2_ref[...] = h * dv


def _tc_mid(p1, dinv, h1, b1):
    grid = (NPAD // BR,)
    return pl.pallas_call(
        _mid_body,
        grid=grid,
        in_specs=[
            pl.BlockSpec((NC, BR, NHID), lambda i: (0, i, 0)),
            pl.BlockSpec((BR,), lambda i: (i,)),
            pl.BlockSpec((BR, NHID), lambda i: (i, 0)),
            pl.BlockSpec((NHID,), lambda i: (0,)),
        ],
        out_specs=[
            pl.BlockSpec((BR, NHID), lambda i: (i, 0)),
            pl.BlockSpec((BR, NHID), lambda i: (i, 0)),
        ],
        out_shape=[
            jax.ShapeDtypeStruct((NPAD, NHID), jnp.float32),
            jax.ShapeDtypeStruct((NPAD, NHID), jnp.float32),
        ],
    )(p1, dinv, h1, b1)


def _fin_body(p_ref, dinv_ref, h_ref, wm_ref, bm_ref, wl_ref, bl_ref,
              mu_ref, lv_ref):
    dv = dinv_ref[...][:, None]
    g = (p_ref[0] + p_ref[1]) * dv + dv * dv * h_ref[...]
    mu_ref[...] = jnp.dot(g, wm_ref[...], preferred_element_type=jnp.float32) + bm_ref[...]
    lv_ref[...] = jnp.dot(g, wl_ref[...], preferred_element_type=jnp.float32) + bl_ref[...]


def _tc_fin(p2, dinv, h, W_mu, b_mu, W_logvar, b_logvar):
    grid = (NPAD // BR,)
    return pl.pallas_call(
        _fin_body,
        grid=grid,
        in_specs=[
            pl.BlockSpec((NC, BR, NHID), lambda i: (0, i, 0)),
            pl.BlockSpec((BR,), lambda i: (i,)),
            pl.BlockSpec((BR, NHID), lambda i: (i, 0)),
            pl.BlockSpec((NHID, LATENT), lambda i: (0, 0)),
            pl.BlockSpec((LATENT,), lambda i: (0,)),
            pl.BlockSpec((NHID, LATENT), lambda i: (0, 0)),
            pl.BlockSpec((LATENT,), lambda i: (0,)),
        ],
        out_specs=[
            pl.BlockSpec((BR, LATENT), lambda i: (i, 0)),
            pl.BlockSpec((BR, LATENT), lambda i: (i, 0)),
        ],
        out_shape=[
            jax.ShapeDtypeStruct((NPAD, LATENT), jnp.float32),
            jax.ShapeDtypeStruct((NPAD, LATENT), jnp.float32),
        ],
    )(p2, dinv, h, W_mu, b_mu, W_logvar, b_logvar)


@jax.jit
def kernel(x, edge_index, W1, b1, W_mu, b_mu, W_logvar, b_logvar):
    src = edge_index[0]
    dst = edge_index[1]
    pad = EPAD - N_EDGES
    src_pad = jnp.arange(pad, dtype=jnp.int32) % N_NODES
    src_p = jnp.concatenate([src, src_pad]).reshape(NW * NCH, CH)
    dst_p = jnp.concatenate([dst, jnp.full((pad,), DUMMY, jnp.int32)]).reshape(
        NW * NCH, CH)
    x_p = jnp.pad(x, ((0, NPAD - N_NODES), (0, 0)))

    zerosD = jnp.zeros((STRIPE, D_FEAT), jnp.float32)

    hist, prop = _sc_kernels()
    onesD = jnp.ones((CH, D_FEAT), jnp.float32)
    deg_parts = hist(dst_p, onesD, zerosD)
    h1, u1, dinv = _tc_mm1(x_p, W1, deg_parts)
    p1 = prop(src_p, dst_p, u1, zerosD)
    h, u2 = _tc_mid(p1, dinv, h1, b1)
    p2 = prop(src_p, dst_p, u2, zerosD)
    mu, logvar = _tc_fin(p2, dinv, h, W_mu, b_mu, W_logvar, b_logvar)
    return mu[:N_NODES], logvar[:N_NODES]
